# Initial kernel scaffold; baseline (speedup 1.0000x reference)
import jax
import jax.numpy as jnp
from jax.experimental import pallas as pl


def kernel(x, embed_mat):
    raise NotImplementedError("write your pallas kernel here")


# speedup vs baseline: 4.5813x; 4.5813x over previous
import functools

import jax
import jax.numpy as jnp
from jax import lax
from jax.experimental import pallas as pl
from jax.experimental.pallas import tpu as pltpu
from jax.experimental.pallas import tpu_sc as plsc

_D = 64
_NC, _NS = 2, 16
_NW = _NC * _NS
_CH = 128
_GRP = 5
_GROWS = _GRP * _CH


def _embed_body(idx_hbm, table_hbm, out_hbm, idx_v, rows_v, gsem):
    wid = lax.axis_index("s") * _NC + lax.axis_index("c")
    nch = idx_v.shape[0]
    ngrp = nch // _GRP
    base_row = wid * (nch * _CH)
    pltpu.sync_copy(idx_hbm.at[wid], idx_v)

    def group(g, carry):
        cps = []
        for c in range(_GRP):
            cps.append(pltpu.async_copy(
                table_hbm.at[idx_v.at[g * _GRP + c]],
                rows_v.at[pl.ds(c * _CH, _CH)],
                gsem))
        for cp in cps:
            cp.wait()
        pltpu.sync_copy(rows_v, out_hbm.at[pl.ds(base_row + g * _GROWS, _GROWS)])
        return carry

    lax.fori_loop(0, ngrp, group, 0)


def kernel(x, embed_mat):
    b, h = x.shape
    tot = b * h
    nch = tot // (_NW * _CH)
    idx3d = x.astype(jnp.int32).reshape(_NW, nch, _CH)
    mesh = plsc.VectorSubcoreMesh(core_axis_name="c", subcore_axis_name="s",
                                  num_cores=_NC, num_subcores=_NS)
    out = pl.kernel(
        _embed_body,
        out_type=jax.ShapeDtypeStruct((tot, _D), jnp.float32),
        mesh=mesh,
        scratch_types=[
            pltpu.VMEM((nch, _CH), jnp.int32),
            pltpu.VMEM((_GROWS, _D), jnp.float32),
            pltpu.SemaphoreType.DMA,
        ],
        compiler_params=pltpu.CompilerParams(use_tc_tiling_on_sc=False),
    )(idx3d, embed_mat)
    return out.reshape(b, h, _D)


# speedup vs baseline: 4.6107x; 1.0064x over previous
import functools

import jax
import jax.numpy as jnp
from jax import lax
from jax.experimental import pallas as pl
from jax.experimental.pallas import tpu as pltpu
from jax.experimental.pallas import tpu_sc as plsc

_D = 64
_NC, _NS = 2, 16
_NW = _NC * _NS
_CH = 128
_GRP = 5
_GROWS = _GRP * _CH


def _embed_body(idx_hbm, table_hbm, out_hbm, idx_v, buf0, buf1, gsem, wsem):
    wid = lax.axis_index("s") * _NC + lax.axis_index("c")
    nch = idx_v.shape[0]
    ngrp = nch // _GRP
    base_row = wid * (nch * _CH)
    bufs = (buf0, buf1)
    pltpu.sync_copy(idx_hbm.at[wid], idx_v)

    def fire(g, buf):
        for c in range(_GRP):
            pltpu.async_copy(table_hbm.at[idx_v.at[g * _GRP + c]],
                             buf.at[pl.ds(c * _CH, _CH)], gsem)

    def drain_gathers(buf):
        for c in range(_GRP):
            pltpu.make_async_copy(table_hbm.at[pl.ds(0, _CH)],
                                  buf.at[pl.ds(c * _CH, _CH)], gsem).wait()

    def fire_write(g, buf):
        pltpu.async_copy(buf, out_hbm.at[pl.ds(base_row + g * _GROWS, _GROWS)],
                         wsem)

    def wait_write(buf):
        pltpu.make_async_copy(table_hbm.at[pl.ds(0, _GROWS)], buf, wsem).wait()

    fire(0, buf0)

    def step(i, carry):
        for b in range(2):
            g = 2 * i + b
            buf, obuf = bufs[b], bufs[1 - b]
            drain_gathers(buf)

            @pl.when(g + 1 < ngrp)
            def _():
                @pl.when(g > 0)
                def _():
                    wait_write(obuf)
                fire(g + 1, obuf)

            fire_write(g, buf)
        return carry

    lax.fori_loop(0, ngrp // 2, step, 0)
    wait_write(bufs[0])
    wait_write(bufs[1])


def kernel(x, embed_mat):
    b, h = x.shape
    tot = b * h
    nch = tot // (_NW * _CH)
    idx3d = x.astype(jnp.int32).reshape(_NW, nch, _CH)
    mesh = plsc.VectorSubcoreMesh(core_axis_name="c", subcore_axis_name="s",
                                  num_cores=_NC, num_subcores=_NS)
    out = pl.kernel(
        _embed_body,
        out_type=jax.ShapeDtypeStruct((tot, _D), jnp.float32),
        mesh=mesh,
        scratch_types=[
            pltpu.VMEM((nch, _CH), jnp.int32),
            pltpu.VMEM((_GROWS, _D), jnp.float32),
            pltpu.VMEM((_GROWS, _D), jnp.float32),
            pltpu.SemaphoreType.DMA,
            pltpu.SemaphoreType.DMA,
        ],
        compiler_params=pltpu.CompilerParams(use_tc_tiling_on_sc=False),
    )(idx3d, embed_mat)
    return out.reshape(b, h, _D)


# speedup vs baseline: 4.6124x; 1.0004x over previous
import functools

import jax
import jax.numpy as jnp
from jax import lax
from jax.experimental import pallas as pl
from jax.experimental.pallas import tpu as pltpu
from jax.experimental.pallas import tpu_sc as plsc

_D = 64
_NC, _NS = 2, 16
_NW = _NC * _NS
_GB = 16


def _embed_body(idx_hbm, table_hbm, out_hbm, idx_v, buf0, buf1, gsem, wsem):
    wid = lax.axis_index("s") * _NC + lax.axis_index("c")
    bpw = idx_v.shape[0]
    hist = idx_v.shape[1]
    ngrp = bpw // _GB
    base_b = wid * bpw
    bufs = (buf0, buf1)
    pltpu.sync_copy(idx_hbm.at[pl.ds(base_b, bpw)], idx_v)

    def fire(g, buf):
        for k in range(_GB):
            pltpu.async_copy(table_hbm.at[idx_v.at[g * _GB + k]],
                             buf.at[k], gsem)

    def drain_gathers(buf):
        for k in range(_GB):
            pltpu.make_async_copy(table_hbm.at[pl.ds(0, hist)],
                                  buf.at[k], gsem).wait()

    def fire_write(g, buf):
        pltpu.async_copy(buf, out_hbm.at[pl.ds(base_b + g * _GB, _GB)], wsem)

    def wait_write(buf):
        pltpu.make_async_copy(out_hbm.at[pl.ds(0, _GB)], buf, wsem).wait()

    fire(0, buf0)

    def step(i, carry):
        for b in range(2):
            g = 2 * i + b
            buf, obuf = bufs[b], bufs[1 - b]
            drain_gathers(buf)

            @pl.when(g + 1 < ngrp)
            def _():
                @pl.when(g > 0)
                def _():
                    wait_write(obuf)
                fire(g + 1, obuf)

            fire_write(g, buf)
        return carry

    lax.fori_loop(0, ngrp // 2, step, 0)
    wait_write(bufs[0])
    wait_write(bufs[1])


def kernel(x, embed_mat):
    b, h = x.shape
    bpw = b // _NW
    mesh = plsc.VectorSubcoreMesh(core_axis_name="c", subcore_axis_name="s",
                                  num_cores=_NC, num_subcores=_NS)
    return pl.kernel(
        _embed_body,
        out_type=jax.ShapeDtypeStruct((b, h, _D), jnp.float32),
        mesh=mesh,
        scratch_types=[
            pltpu.VMEM((bpw, h), jnp.int32),
            pltpu.VMEM((_GB, h, _D), jnp.float32),
            pltpu.VMEM((_GB, h, _D), jnp.float32),
            pltpu.SemaphoreType.DMA,
            pltpu.SemaphoreType.DMA,
        ],
        compiler_params=pltpu.CompilerParams(use_tc_tiling_on_sc=False),
    )(x.astype(jnp.int32), embed_mat)


# speedup vs baseline: 6.8952x; 1.4949x over previous
import functools

import jax
import jax.numpy as jnp
from jax import lax
from jax.experimental import pallas as pl
from jax.experimental.pallas import tpu as pltpu
from jax.experimental.pallas import tpu_sc as plsc

_D = 64
_NC, _NS = 2, 16
_NW = _NC * _NS
_GB = 16
_HP = 56
_DP = 128


def _embed_body(idx_hbm, table_hbm, out_hbm, idx_v, buf0, buf1, gsem, wsem):
    wid = lax.axis_index("s") * _NC + lax.axis_index("c")
    bpw = idx_v.shape[0]
    hist = idx_v.shape[1]
    ngrp = bpw // _GB
    base_b = wid * bpw
    bufs = (buf0, buf1)
    pltpu.sync_copy(idx_hbm.at[pl.ds(base_b, bpw)], idx_v)

    def fire(g, buf):
        for k in range(_GB):
            pltpu.async_copy(table_hbm.at[idx_v.at[g * _GB + k]],
                             buf.at[k], gsem)

    def drain_gathers(buf):
        for k in range(_GB):
            pltpu.make_async_copy(table_hbm.at[pl.ds(0, hist)],
                                  buf.at[k], gsem).wait()

    def fire_write(g, buf):
        pltpu.async_copy(buf,
                         out_hbm.at[pl.ds(base_b + g * _GB, _GB),
                                    pl.ds(0, hist), pl.ds(0, _D)], wsem)

    def wait_write(buf):
        pltpu.make_async_copy(out_hbm.at[pl.ds(0, _GB),
                                         pl.ds(0, hist), pl.ds(0, _D)],
                              buf, wsem).wait()

    fire(0, buf0)

    def step(i, carry):
        for b in range(2):
            g = 2 * i + b
            buf, obuf = bufs[b], bufs[1 - b]
            drain_gathers(buf)

            @pl.when(g + 1 < ngrp)
            def _():
                @pl.when(g > 0)
                def _():
                    wait_write(obuf)
                fire(g + 1, obuf)

            fire_write(g, buf)
        return carry

    lax.fori_loop(0, ngrp // 2, step, 0)
    wait_write(bufs[0])
    wait_write(bufs[1])


def kernel(x, embed_mat):
    b, h = x.shape
    bpw = b // _NW
    mesh = plsc.VectorSubcoreMesh(core_axis_name="c", subcore_axis_name="s",
                                  num_cores=_NC, num_subcores=_NS)
    y3 = pl.kernel(
        _embed_body,
        out_type=jax.ShapeDtypeStruct((b, _HP, _DP), jnp.float32),
        mesh=mesh,
        scratch_types=[
            pltpu.VMEM((bpw, h), jnp.int32),
            pltpu.VMEM((_GB, h, _D), jnp.float32),
            pltpu.VMEM((_GB, h, _D), jnp.float32),
            pltpu.SemaphoreType.DMA,
            pltpu.SemaphoreType.DMA,
        ],
        compiler_params=pltpu.CompilerParams(use_tc_tiling_on_sc=False),
    )(x.astype(jnp.int32), embed_mat)
    return y3[:, :h, :_D]


# speedup vs baseline: 7.0243x; 1.0187x over previous
import functools

import jax
import jax.numpy as jnp
from jax import lax
from jax.experimental import pallas as pl
from jax.experimental.pallas import tpu as pltpu
from jax.experimental.pallas import tpu_sc as plsc

_D = 64
_NC, _NS = 2, 16
_NW = _NC * _NS
_GB = 8
_NB = 4
_HP = 56
_DP = 128


def _embed_body(idx_hbm, table_hbm, out_hbm, idx_v, *rest):
    bufs = rest[:_NB]
    gsems = rest[_NB:2 * _NB]
    wsems = rest[2 * _NB:3 * _NB]
    wid = lax.axis_index("s") * _NC + lax.axis_index("c")
    bpw = idx_v.shape[0]
    hist = idx_v.shape[1]
    ngrp = bpw // _GB
    base_b = wid * bpw
    pltpu.sync_copy(idx_hbm.at[pl.ds(base_b, bpw)], idx_v)

    def fire(g, b):
        for k in range(_GB):
            pltpu.async_copy(table_hbm.at[idx_v.at[g * _GB + k]],
                             bufs[b].at[k], gsems[b])

    def drain_gathers(b):
        pltpu.make_async_copy(out_hbm.at[pl.ds(0, _GB),
                                         pl.ds(0, hist), pl.ds(0, _D)],
                              bufs[b], gsems[b]).wait()

    def fire_write(g, b):
        pltpu.async_copy(bufs[b],
                         out_hbm.at[pl.ds(base_b + g * _GB, _GB),
                                    pl.ds(0, hist), pl.ds(0, _D)], wsems[b])

    def wait_write(b):
        pltpu.make_async_copy(out_hbm.at[pl.ds(0, _GB),
                                         pl.ds(0, hist), pl.ds(0, _D)],
                              bufs[b], wsems[b]).wait()

    for g in range(_NB - 1):
        fire(g, g)

    def step(i, carry):
        for b in range(_NB):
            g = _NB * i + b
            drain_gathers(b)
            fire_write(g, b)
            j = g + _NB - 1
            jb = (_NB - 1 + b) % _NB
            @pl.when(j < ngrp)
            def _():
                @pl.when(j >= _NB)
                def _():
                    wait_write(jb)
                fire(j, jb)
        return carry

    lax.fori_loop(0, ngrp // _NB, step, 0)
    for b in range(_NB):
        wait_write(b)


def kernel(x, embed_mat):
    b, h = x.shape
    bpw = b // _NW
    mesh = plsc.VectorSubcoreMesh(core_axis_name="c", subcore_axis_name="s",
                                  num_cores=_NC, num_subcores=_NS)
    y3 = pl.kernel(
        _embed_body,
        out_type=jax.ShapeDtypeStruct((b, _HP, _DP), jnp.float32),
        mesh=mesh,
        scratch_types=[
            pltpu.VMEM((bpw, h), jnp.int32),
            *[pltpu.VMEM((_GB, h, _D), jnp.float32) for _ in range(_NB)],
            *[pltpu.SemaphoreType.DMA for _ in range(2 * _NB)],
        ],
        compiler_params=pltpu.CompilerParams(use_tc_tiling_on_sc=False),
    )(x.astype(jnp.int32), embed_mat)
    return y3[:, :h, :_D]
